# Initial kernel scaffold; baseline (speedup 1.0000x reference)
import jax
import jax.numpy as jnp
from jax.experimental import pallas as pl


def kernel(x, edge_index, W, a):
    raise NotImplementedError("write your pallas kernel here")


# speedup vs baseline: 21.4404x; 21.4404x over previous
import functools

import jax
import jax.numpy as jnp
from jax import lax
from jax.experimental import pallas as pl
from jax.experimental.pallas import tpu as pltpu
from jax.experimental.pallas import tpu_sc as plsc

N = 10000
E = 320000
D = 128
ALPHA = 0.2

NC = 2
NS = 16
NW = NC * NS
EPT = E // NW
CH = 80
NCH = EPT // CH
N2 = 10240
RPT = N2 // NS
ZR = 128

_f32 = jnp.float32
_i32 = jnp.int32


def _prep_body(x_ref, w_ref, a_ref, hp_ref, s_ref):
    h = jnp.dot(x_ref[...], w_ref[...], preferred_element_type=_f32)
    a2 = a_ref[...].reshape(2, D)
    s = lax.dot_general(h, a2, (((1,), (1,)), ((), ())),
                        preferred_element_type=_f32)
    s_ref[...] = s
    hp_ref[...] = h


def _tc_prep(x, W, a):
    blk = 1000
    grid = N // blk
    return pl.pallas_call(
        _prep_body,
        grid=(grid,),
        in_specs=[
            pl.BlockSpec((blk, D), lambda i: (i, 0)),
            pl.BlockSpec((D, D), lambda i: (0, 0)),
            pl.BlockSpec((1, 2 * D), lambda i: (0, 0)),
        ],
        out_specs=[
            pl.BlockSpec((blk, D), lambda i: (i, 0)),
            pl.BlockSpec((blk, 2), lambda i: (i, 0)),
        ],
        out_shape=[
            jax.ShapeDtypeStruct((N, D), _f32),
            jax.ShapeDtypeStruct((N, 2), _f32),
        ],
    )(x, W, a)


def _sc_body(eidx_hbm, s1_hbm, s2_hbm, hp_hbm, u_out, dn_out,
             src_v, dst_v, sv1, sv2, pbuf, rows, zbuf, zdn, u_sh, dn_sh, sem):
    cid = lax.axis_index("c")
    sid = lax.axis_index("s")
    wid = cid * NS + sid

    def _zb(i, _):
        for c in range(D // 16):
            zbuf[i, pl.ds(c * 16, 16)] = jnp.zeros((16,), _f32)
        return 0
    lax.fori_loop(0, ZR, _zb, 0)
    for k in range(RPT // 16):
        zdn[pl.ds(k * 16, 16)] = jnp.zeros((16,), _f32)
    for k in range(RPT // ZR):
        pltpu.sync_copy(zbuf, u_sh.at[pl.ds(sid * RPT + k * ZR, ZR)])
    pltpu.sync_copy(zdn, dn_sh.at[pl.ds(sid * RPT, RPT)])
    plsc.subcore_barrier()

    pltpu.sync_copy(eidx_hbm.at[0, wid], src_v)
    pltpu.sync_copy(eidx_hbm.at[1, wid], dst_v)

    def _chunk(j, _):
        src_c = src_v.at[j]
        dst_c = dst_v.at[j]
        c1 = pltpu.async_copy(s1_hbm.at[src_c], sv1, sem)
        c2 = pltpu.async_copy(s2_hbm.at[dst_c], sv2, sem)
        c3 = pltpu.async_copy(hp_hbm.at[dst_c], rows, sem)
        c1.wait()
        c2.wait()
        for k in range(CH // 16):
            sl = pl.ds(k * 16, 16)
            e = sv1[sl] + sv2[sl]
            e = jnp.where(e > 0, e, ALPHA * e)
            pbuf[sl] = jnp.exp(e)
        pltpu.sync_copy(pbuf, dn_sh.at[src_c], add=True)
        c3.wait()

        def _row(r, _):
            pb = plsc.load_gather(pbuf, [jnp.full((16,), r, _i32)])
            for c in range(D // 16):
                sl = pl.ds(c * 16, 16)
                rows[r, sl] = rows[r, sl] * pb
            return 0
        lax.fori_loop(0, CH, _row, 0)

        pltpu.sync_copy(rows, u_sh.at[src_c], add=True)
        return 0

    lax.fori_loop(0, NCH, _chunk, 0)
    plsc.subcore_barrier()

    pltpu.sync_copy(u_sh.at[pl.ds(sid * RPT, RPT)],
                    u_out.at[cid, pl.ds(sid * RPT, RPT)])
    pltpu.sync_copy(dn_sh.at[pl.ds(sid * RPT, RPT)],
                    dn_out.at[cid, pl.ds(sid * RPT, RPT)])


def _sc_agg(eidx, s1, s2, hp):
    mesh = plsc.VectorSubcoreMesh(core_axis_name="c", subcore_axis_name="s")
    return pl.kernel(
        _sc_body,
        out_type=(jax.ShapeDtypeStruct((NC, N2, D), _f32),
                  jax.ShapeDtypeStruct((NC, N2), _f32)),
        mesh=mesh,
        compiler_params=pltpu.CompilerParams(
            needs_layout_passes=False, use_tc_tiling_on_sc=False),
        scratch_types=[
            pltpu.VMEM((NCH, CH), _i32),
            pltpu.VMEM((NCH, CH), _i32),
            pltpu.VMEM((CH,), _f32),
            pltpu.VMEM((CH,), _f32),
            pltpu.VMEM((CH,), _f32),
            pltpu.VMEM((CH, D), _f32),
            pltpu.VMEM((ZR, D), _f32),
            pltpu.VMEM((RPT,), _f32),
            pltpu.VMEM_SHARED((N2, D), _f32),
            pltpu.VMEM_SHARED((N2,), _f32),
            pltpu.SemaphoreType.DMA,
        ],
    )(eidx, s1, s2, hp)


def _combine_body(u_ref, dn_ref, o_ref):
    num = u_ref[0] + u_ref[1]
    den = dn_ref[0] + dn_ref[1]
    den = jnp.where(den > 0, den, 1.0)
    r = num / den[:, None]
    o_ref[...] = jnp.where(r > 0, r, jnp.exp(jnp.minimum(r, 0.0)) - 1.0)


def _tc_combine(u, dn):
    blk = 1024
    grid = N2 // blk
    return pl.pallas_call(
        _combine_body,
        grid=(grid,),
        in_specs=[
            pl.BlockSpec((NC, blk, D), lambda i: (0, i, 0)),
            pl.BlockSpec((NC, blk), lambda i: (0, i)),
        ],
        out_specs=pl.BlockSpec((blk, D), lambda i: (i, 0)),
        out_shape=jax.ShapeDtypeStruct((N2, D), _f32),
    )(u, dn)


def kernel(x, edge_index, W, a):
    hp, s = _tc_prep(x, W, a)
    s1 = s[:, 0]
    s2 = s[:, 1]
    eidx = edge_index.reshape(2, NW, NCH, CH)
    u, dn = _sc_agg(eidx, s1, s2, hp)
    return _tc_combine(u, dn)[:N]


# speedup vs baseline: 38.2056x; 1.7819x over previous
import functools

import jax
import jax.numpy as jnp
from jax import lax
from jax.experimental import pallas as pl
from jax.experimental.pallas import tpu as pltpu
from jax.experimental.pallas import tpu_sc as plsc

N = 10000
E = 320000
D = 128
ALPHA = 0.2

NC = 2
NS = 16
NW = NC * NS
EPT = E // NW
CH = 80
NCH = EPT // CH
N2 = 10240
RPT = N2 // NS
ZR = 16

_f32 = jnp.float32
_i32 = jnp.int32


def _prep_body(x_ref, w_ref, a_ref, hp_ref, s_ref):
    h = jnp.dot(x_ref[...], w_ref[...], preferred_element_type=_f32)
    a2 = a_ref[...].reshape(2, D)
    s = lax.dot_general(h, a2, (((1,), (1,)), ((), ())),
                        preferred_element_type=_f32)
    s_ref[...] = s
    hp_ref[...] = h


def _tc_prep(x, W, a):
    blk = 1000
    grid = N // blk
    return pl.pallas_call(
        _prep_body,
        grid=(grid,),
        in_specs=[
            pl.BlockSpec((blk, D), lambda i: (i, 0)),
            pl.BlockSpec((D, D), lambda i: (0, 0)),
            pl.BlockSpec((1, 2 * D), lambda i: (0, 0)),
        ],
        out_specs=[
            pl.BlockSpec((blk, D), lambda i: (i, 0)),
            pl.BlockSpec((blk, 2), lambda i: (i, 0)),
        ],
        out_shape=[
            jax.ShapeDtypeStruct((N, D), _f32),
            jax.ShapeDtypeStruct((N, 2), _f32),
        ],
    )(x, W, a)


def _sc_body(eidx_hbm, s1_hbm, s2_hbm, hp_hbm, u_out, dn_out,
             src_v, dst_v, sv1, sv2, pbuf, rows, zbuf, zdn,
             u_sh, dn_sh, gsem, ssem, dsem0, dsem1, sems):
    cid = lax.axis_index("c")
    sid = lax.axis_index("s")
    wid = cid * NS + sid

    for c in range(D // 16):
        for i in range(ZR):
            zbuf[i, pl.ds(c * 16, 16)] = jnp.zeros((16,), _f32)
    for k in range(RPT // 16):
        zdn[pl.ds(k * 16, 16)] = jnp.zeros((16,), _f32)
    for k in range(RPT // ZR):
        pltpu.sync_copy(zbuf, u_sh.at[pl.ds(sid * RPT + k * ZR, ZR)])
    pltpu.sync_copy(zdn, dn_sh.at[pl.ds(sid * RPT, RPT)])

    pltpu.sync_copy(eidx_hbm.at[0, wid], src_v)
    pltpu.sync_copy(eidx_hbm.at[1, wid], dst_v)
    plsc.subcore_barrier()

    pltpu.async_copy(s1_hbm.at[src_v.at[0]], sv1.at[0], sems)
    pltpu.async_copy(s2_hbm.at[dst_v.at[0]], sv2.at[0], sems)
    pltpu.async_copy(hp_hbm.at[dst_v.at[0]], rows.at[0], gsem)

    def _chunk(j, _):
        b = lax.rem(j, 2)
        pltpu.make_async_copy(s1_hbm.at[src_v.at[j]], sv1.at[b], sems).wait()
        pltpu.make_async_copy(s2_hbm.at[dst_v.at[j]], sv2.at[b], sems).wait()

        @pl.when(j + 1 < NCH)
        def _():
            pltpu.async_copy(s1_hbm.at[src_v.at[j + 1]], sv1.at[1 - b], sems)
            pltpu.async_copy(s2_hbm.at[dst_v.at[j + 1]], sv2.at[1 - b], sems)

        @pl.when((j >= 2) & (b == 0))
        def _():
            pltpu.make_async_copy(pbuf.at[b], dn_sh.at[src_v.at[j - 2]],
                                  dsem0).wait()

        @pl.when((j >= 2) & (b == 1))
        def _():
            pltpu.make_async_copy(pbuf.at[b], dn_sh.at[src_v.at[j - 2]],
                                  dsem1).wait()

        for m in range(CH // 16):
            sl = pl.ds(m * 16, 16)
            e = sv1[b, sl] + sv2[b, sl]
            e = jnp.where(e > 0, e, ALPHA * e)
            pbuf[b, sl] = jnp.exp(e)

        @pl.when(b == 0)
        def _():
            pltpu.async_copy(pbuf.at[b], dn_sh.at[src_v.at[j]], dsem0,
                             add=True)

        @pl.when(b == 1)
        def _():
            pltpu.async_copy(pbuf.at[b], dn_sh.at[src_v.at[j]], dsem1,
                             add=True)

        pltpu.make_async_copy(hp_hbm.at[dst_v.at[j]], rows.at[b], gsem).wait()

        @pl.when(j >= 1)
        def _():
            pltpu.make_async_copy(rows.at[1 - b],
                                  u_sh.at[src_v.at[j - 1]], ssem).wait()

        @pl.when(j + 1 < NCH)
        def _():
            pltpu.async_copy(hp_hbm.at[dst_v.at[j + 1]], rows.at[1 - b], gsem)

        @plsc.parallel_loop(0, CH, 1, unroll=4)
        def _scale(r):
            pb = plsc.load_gather(pbuf, [jnp.full((16,), b, _i32),
                                         jnp.full((16,), r, _i32)])
            for c in range(D // 16):
                sl = pl.ds(c * 16, 16)
                rows[b, r, sl] = rows[b, r, sl] * pb

        pltpu.async_copy(rows.at[b], u_sh.at[src_v.at[j]], ssem, add=True)
        return 0

    lax.fori_loop(0, NCH, _chunk, 0)
    pltpu.make_async_copy(pbuf.at[0], dn_sh.at[src_v.at[0]], dsem0).wait()
    pltpu.make_async_copy(pbuf.at[1], dn_sh.at[src_v.at[0]], dsem1).wait()
    pltpu.make_async_copy(rows.at[(NCH - 1) % 2],
                          u_sh.at[src_v.at[NCH - 1]], ssem).wait()
    plsc.subcore_barrier()

    pltpu.sync_copy(u_sh.at[pl.ds(sid * RPT, RPT)],
                    u_out.at[cid, pl.ds(sid * RPT, RPT)])
    pltpu.sync_copy(dn_sh.at[pl.ds(sid * RPT, RPT)],
                    dn_out.at[cid, pl.ds(sid * RPT, RPT)])


def _sc_agg(eidx, s1, s2, hp):
    mesh = plsc.VectorSubcoreMesh(core_axis_name="c", subcore_axis_name="s")
    return pl.kernel(
        _sc_body,
        out_type=(jax.ShapeDtypeStruct((NC, N2, D), _f32),
                  jax.ShapeDtypeStruct((NC, N2), _f32)),
        mesh=mesh,
        compiler_params=pltpu.CompilerParams(
            needs_layout_passes=False, use_tc_tiling_on_sc=False),
        scratch_types=[
            pltpu.VMEM((NCH, CH), _i32),
            pltpu.VMEM((NCH, CH), _i32),
            pltpu.VMEM((2, CH), _f32),
            pltpu.VMEM((2, CH), _f32),
            pltpu.VMEM((2, CH), _f32),
            pltpu.VMEM((2, CH, D), _f32),
            pltpu.VMEM((ZR, D), _f32),
            pltpu.VMEM((RPT,), _f32),
            pltpu.VMEM_SHARED((N2, D), _f32),
            pltpu.VMEM_SHARED((N2,), _f32),
            pltpu.SemaphoreType.DMA,
            pltpu.SemaphoreType.DMA,
            pltpu.SemaphoreType.DMA,
            pltpu.SemaphoreType.DMA,
            pltpu.SemaphoreType.DMA,
        ],
    )(eidx, s1, s2, hp)


def _combine_body(u_ref, dn_ref, o_ref):
    num = u_ref[0] + u_ref[1]
    den = dn_ref[0] + dn_ref[1]
    den = jnp.where(den > 0, den, 1.0)
    r = num / den[:, None]
    o_ref[...] = jnp.where(r > 0, r, jnp.exp(jnp.minimum(r, 0.0)) - 1.0)


def _tc_combine(u, dn):
    blk = 1024
    grid = N2 // blk
    return pl.pallas_call(
        _combine_body,
        grid=(grid,),
        in_specs=[
            pl.BlockSpec((NC, blk, D), lambda i: (0, i, 0)),
            pl.BlockSpec((NC, blk), lambda i: (0, i)),
        ],
        out_specs=pl.BlockSpec((blk, D), lambda i: (i, 0)),
        out_shape=jax.ShapeDtypeStruct((N2, D), _f32),
    )(u, dn)


def kernel(x, edge_index, W, a):
    hp, s = _tc_prep(x, W, a)
    s1 = s[:, 0]
    s2 = s[:, 1]
    eidx = edge_index.reshape(2, NW, NCH, CH)
    u, dn = _sc_agg(eidx, s1, s2, hp)
    return _tc_combine(u, dn)[:N]
